# baseline (speedup 1.0000x reference)
import functools

import jax
import jax.numpy as jnp
from jax.experimental import pallas as pl
from jax.experimental.pallas import tpu as pltpu

_EPS = 1e-8


def _scan_body(w_ref, m_ref, kt_ref, max_ref, idx_ref, *, lanes, n):
    i = pl.program_id(0)
    b = kt_ref[...]
    dn = (((1,), (0,)), ((), ()))
    dots = jax.lax.dot_general(w_ref[...], b, dn,
                               preferred_element_type=jnp.float32)
    sumsq = jax.lax.dot_general(m_ref[...], b * b, dn,
                                preferred_element_type=jnp.float32)
    scores = dots / jnp.maximum(jnp.sqrt(sumsq), _EPS)
    gidx = jax.lax.broadcasted_iota(jnp.int32, scores.shape, 1) + i * lanes
    scores = jnp.where(gidx < n, scores, -jnp.inf)
    lm = jnp.max(scores)
    la = jnp.min(jnp.where(scores == lm, gidx, jnp.int32(2147483647)))
    max_ref[...] = jnp.full(max_ref.shape, lm, jnp.float32)
    idx_ref[...] = jnp.full(idx_ref.shape, la, jnp.int32)


def _merge_body(max_ref, idx_ref, vals_ref, out_ref, sem):
    maxs = max_ref[...]
    idxs = idx_ref[...]
    gmax = jnp.max(maxs)
    best = jnp.min(jnp.where(maxs == gmax, idxs, jnp.int32(2147483647)))
    cp = pltpu.make_async_copy(vals_ref.at[pl.ds(best, 1), :], out_ref, sem)
    cp.start()
    cp.wait()


def kernel(key, keys, vals):
    n, d = keys.shape
    kt = keys.T
    lanes = 64000 if n >= 64000 else n
    steps = (n + lanes - 1) // lanes

    w = key.reshape(1, d).astype(jnp.float32)
    m = jnp.ones((1, d), jnp.float32)

    maxs, idxs = pl.pallas_call(
        functools.partial(_scan_body, lanes=lanes, n=n),
        grid=(steps,),
        in_specs=[
            pl.BlockSpec((1, d), lambda i: (0, 0)),
            pl.BlockSpec((1, d), lambda i: (0, 0)),
            pl.BlockSpec((d, lanes), lambda i: (0, i)),
        ],
        out_specs=[
            pl.BlockSpec((1, 1, 128), lambda i: (i, 0, 0)),
            pl.BlockSpec((1, 1, 128), lambda i: (i, 0, 0)),
        ],
        out_shape=[
            jax.ShapeDtypeStruct((steps, 1, 128), jnp.float32),
            jax.ShapeDtypeStruct((steps, 1, 128), jnp.int32),
        ],
        compiler_params=pltpu.CompilerParams(
            dimension_semantics=("parallel",)),
    )(w, m, kt)

    out = pl.pallas_call(
        _merge_body,
        in_specs=[
            pl.BlockSpec(memory_space=pltpu.MemorySpace.VMEM),
            pl.BlockSpec(memory_space=pltpu.MemorySpace.VMEM),
            pl.BlockSpec(memory_space=pltpu.MemorySpace.HBM),
        ],
        out_specs=pl.BlockSpec(memory_space=pltpu.MemorySpace.HBM),
        out_shape=jax.ShapeDtypeStruct((1, vals.shape[1]), jnp.float32),
        scratch_shapes=[pltpu.SemaphoreType.DMA],
    )(maxs, idxs, vals)
    return out.reshape(vals.shape[1])


# speedup vs baseline: 1.0031x; 1.0031x over previous
import functools

import jax
import jax.numpy as jnp
from jax.experimental import pallas as pl
from jax.experimental.pallas import tpu as pltpu

_EPS = 1e-8
_L = 64000
_NSPLIT = 4


def _main_body(w_ref, m_ref, kt_ref, vals_ref, out_ref, buf, sems, gsem,
               *, n, d):
    num_chunks = (n + _L - 1) // _L
    rsub = d // _NSPLIT

    tail_base = ((n - _L) // 128) * 128

    def chunk_copies(c, slot):
        base = pl.multiple_of(jnp.minimum(c * _L, tail_base), 128)
        return [pltpu.make_async_copy(
            kt_ref.at[pl.ds(s * rsub, rsub), pl.ds(base, _L)],
            buf.at[slot, pl.ds(s * rsub, rsub), :],
            sems.at[slot, s]) for s in range(_NSPLIT)]

    for cp in chunk_copies(0, 0):
        cp.start()

    w = w_ref[...]
    m = m_ref[...]
    dn = (((1,), (0,)), ((), ()))

    def loop(c, carry):
        best_s, best_i = carry
        slot = jax.lax.rem(c, 2)

        @pl.when(c + 1 < num_chunks)
        def _prefetch():
            for cp in chunk_copies(c + 1, 1 - slot):
                cp.start()

        for cp in chunk_copies(c, slot):
            cp.wait()
        b = buf[slot]
        dots = jax.lax.dot_general(w, b, dn,
                                   preferred_element_type=jnp.float32)
        sumsq = jax.lax.dot_general(m, b * b, dn,
                                    preferred_element_type=jnp.float32)
        scores = dots / jnp.maximum(jnp.sqrt(sumsq), _EPS)
        base = jnp.minimum(c * _L, ((n - _L) // 128) * 128)
        gidx = jax.lax.broadcasted_iota(jnp.int32, scores.shape, 1) + base
        lm = jnp.max(scores)
        la = jnp.min(jnp.where(scores == lm, gidx, jnp.int32(2147483647)))
        pick = lm > best_s
        return (jnp.where(pick, lm, best_s), jnp.where(pick, la, best_i))

    _, best_i = jax.lax.fori_loop(
        0, num_chunks, loop, (jnp.float32(-jnp.inf), jnp.int32(0)))

    cp = pltpu.make_async_copy(vals_ref.at[pl.ds(best_i, 1), :], out_ref,
                               gsem)
    cp.start()
    cp.wait()


def _small_body(w_ref, m_ref, kt_ref, vals_ref, out_ref, sem, *, n):
    b = kt_ref[...]
    dn = (((1,), (0,)), ((), ()))
    dots = jax.lax.dot_general(w_ref[...], b, dn,
                               preferred_element_type=jnp.float32)
    sumsq = jax.lax.dot_general(m_ref[...], b * b, dn,
                                preferred_element_type=jnp.float32)
    scores = dots / jnp.maximum(jnp.sqrt(sumsq), _EPS)
    gidx = jax.lax.broadcasted_iota(jnp.int32, scores.shape, 1)
    lm = jnp.max(scores)
    la = jnp.min(jnp.where(scores == lm, gidx, jnp.int32(2147483647)))
    cp = pltpu.make_async_copy(vals_ref.at[pl.ds(la, 1), :], out_ref, sem)
    cp.start()
    cp.wait()


def kernel(key, keys, vals):
    n, d = keys.shape
    kt = keys.T
    w = key.reshape(1, d).astype(jnp.float32)
    m = jnp.ones((1, d), jnp.float32)

    if n >= _L and d % _NSPLIT == 0:
        out = pl.pallas_call(
            functools.partial(_main_body, n=n, d=d),
            in_specs=[
                pl.BlockSpec(memory_space=pltpu.MemorySpace.VMEM),
                pl.BlockSpec(memory_space=pltpu.MemorySpace.VMEM),
                pl.BlockSpec(memory_space=pltpu.MemorySpace.HBM),
                pl.BlockSpec(memory_space=pltpu.MemorySpace.HBM),
            ],
            out_specs=pl.BlockSpec(memory_space=pltpu.MemorySpace.HBM),
            out_shape=jax.ShapeDtypeStruct((1, vals.shape[1]), jnp.float32),
            scratch_shapes=[
                pltpu.VMEM((2, d, _L), jnp.float32),
                pltpu.SemaphoreType.DMA((2, _NSPLIT)),
                pltpu.SemaphoreType.DMA,
            ],
        )(w, m, kt, vals)
        return out.reshape(vals.shape[1])

    out = pl.pallas_call(
        functools.partial(_small_body, n=n),
        in_specs=[
            pl.BlockSpec(memory_space=pltpu.MemorySpace.VMEM),
            pl.BlockSpec(memory_space=pltpu.MemorySpace.VMEM),
            pl.BlockSpec(memory_space=pltpu.MemorySpace.VMEM),
            pl.BlockSpec(memory_space=pltpu.MemorySpace.HBM),
        ],
        out_specs=pl.BlockSpec(memory_space=pltpu.MemorySpace.HBM),
        out_shape=jax.ShapeDtypeStruct((1, vals.shape[1]), jnp.float32),
        scratch_shapes=[pltpu.SemaphoreType.DMA],
    )(w, m, kt, vals)
    return out.reshape(vals.shape[1])


# speedup vs baseline: 1.0035x; 1.0004x over previous
import functools

import jax
import jax.numpy as jnp
from jax.experimental import pallas as pl
from jax.experimental.pallas import tpu as pltpu

_EPS = 1e-8


def _body(w_ref, m_ref, kt_ref, vals_ref, out_ref, best_s_ref, best_i_ref,
          sem, *, lanes, n):
    i = pl.program_id(0)

    @pl.when(i == 0)
    def _init():
        best_s_ref[0] = -jnp.inf
        best_i_ref[0] = 0

    b = kt_ref[...]
    dn = (((1,), (0,)), ((), ()))
    dots = jax.lax.dot_general(w_ref[...], b, dn,
                               preferred_element_type=jnp.float32)
    sumsq = jax.lax.dot_general(m_ref[...], b * b, dn,
                                preferred_element_type=jnp.float32)
    scores = dots / jnp.maximum(jnp.sqrt(sumsq), _EPS)
    gidx = jax.lax.broadcasted_iota(jnp.int32, scores.shape, 1) + i * lanes
    scores = jnp.where(gidx < n, scores, -jnp.inf)
    local_max = jnp.max(scores)
    local_arg = jnp.min(jnp.where(scores == local_max, gidx,
                                  jnp.int32(2147483647)))

    @pl.when(local_max > best_s_ref[0])
    def _update():
        best_s_ref[0] = local_max
        best_i_ref[0] = local_arg

    @pl.when(i == pl.num_programs(0) - 1)
    def _gather():
        idx = best_i_ref[0]
        cp = pltpu.make_async_copy(vals_ref.at[pl.ds(idx, 1), :], out_ref, sem)
        cp.start()
        cp.wait()


def _pick_lanes(n):
    if n <= 65536:
        return n
    return 64000


def kernel(key, keys, vals):
    n, d = keys.shape
    lanes = _pick_lanes(n)
    kt = keys.T

    w = key.reshape(1, d).astype(jnp.float32)
    m = jnp.ones((1, d), jnp.float32)

    out = pl.pallas_call(
        functools.partial(_body, lanes=lanes, n=n),
        grid=((n + lanes - 1) // lanes,),
        in_specs=[
            pl.BlockSpec((1, d), lambda i: (0, 0)),
            pl.BlockSpec((1, d), lambda i: (0, 0)),
            pl.BlockSpec((d, lanes), lambda i: (0, i)),
            pl.BlockSpec(memory_space=pltpu.MemorySpace.HBM),
        ],
        out_specs=pl.BlockSpec(memory_space=pltpu.MemorySpace.HBM),
        out_shape=jax.ShapeDtypeStruct((1, vals.shape[1]), jnp.float32),
        scratch_shapes=[
            pltpu.SMEM((1,), jnp.float32),
            pltpu.SMEM((1,), jnp.int32),
            pltpu.SemaphoreType.DMA,
        ],
    )(w, m, kt, vals)
    return out.reshape(vals.shape[1])
